# baseline (speedup 1.0000x reference)
import functools
import math

import jax
import jax.numpy as jnp
from jax import lax
from jax.experimental import pallas as pl
from jax.experimental.pallas import tpu as pltpu
from jax.experimental.pallas import tpu_sc as plsc

D = 128
L = 16
NC = 2
NS = 16
NW = NC * NS
SUB = 128
NBUF = 4
SCALE = float(math.sqrt(float(D)))


@jax.jit
def _embed(x, table):
    R, C = x.shape
    rows_per_w = R // NW
    n_chunks = rows_per_w
    b_per_w = rows_per_w * C

    mesh = plsc.VectorSubcoreMesh(core_axis_name="c", subcore_axis_name="s")

    scratch = [
        pltpu.VMEM((rows_per_w, C), jnp.int32),
    ]
    scratch += [pltpu.VMEM((C, D), jnp.float32) for _ in range(NBUF)]
    scratch += [pltpu.SemaphoreType.DMA for _ in range(2 * NBUF)]

    @functools.partial(
        pl.kernel,
        mesh=mesh,
        out_type=jax.ShapeDtypeStruct((R, C, D), jnp.float32),
        scratch_types=scratch,
    )
    def k(x_hbm, table_hbm, out_hbm, idx_v, *bufs_and_sems):
        rows = bufs_and_sems[:NBUF]
        gsem = bufs_and_sems[NBUF:2 * NBUF]
        ssem = bufs_and_sems[2 * NBUF:3 * NBUF]

        wid = lax.axis_index("s") * NC + lax.axis_index("c")
        xrow0 = wid * rows_per_w

        pltpu.sync_copy(x_hbm.at[pl.ds(xrow0, rows_per_w)], idx_v)

        def gather(g, b):
            return (
                pltpu.make_async_copy(
                    table_hbm.at[idx_v.at[g, pl.ds(0, SUB)]],
                    rows[b].at[pl.ds(0, SUB)],
                    gsem[b]),
                pltpu.make_async_copy(
                    table_hbm.at[idx_v.at[g, pl.ds(SUB, C - SUB)]],
                    rows[b].at[pl.ds(SUB, C - SUB)],
                    gsem[b]),
            )

        def store(g, b):
            return pltpu.make_async_copy(
                rows[b], out_hbm.at[xrow0 + g], ssem[b])

        def start_gather(g, b):
            d0, d1 = gather(g, b)
            d0.start()
            d1.start()

        def wait_gather(g, b):
            d0, d1 = gather(g, b)
            d0.wait()
            d1.wait()

        for b in range(NBUF - 2):
            start_gather(b, b)

        def round_body(go, carry):
            for b in range(NBUF):
                g = go * NBUF + b
                bp = (b - 2) % NBUF

                @pl.when(g + NBUF - 2 < n_chunks)
                def _():
                    @pl.when(g >= 2)
                    def _():
                        store(lax.max(g - 2, 0), bp).wait()
                    start_gather(g + NBUF - 2, bp)

                wait_gather(g, b)

                def scale_quad(i, c2):
                    for r in range(4):
                        for v in range(D // L):
                            rows[b][i * 4 + r, pl.ds(v * L, L)] = (
                                rows[b][i * 4 + r, pl.ds(v * L, L)] * SCALE
                            )
                    return c2

                lax.fori_loop(0, C // 4, scale_quad, 0)
                store(g, b).start()
            return carry

        lax.fori_loop(0, n_chunks // NBUF, round_body, 0)

        for b in range(NBUF):
            store(n_chunks - NBUF + b, b).wait()

    return k(x, table)


def kernel(x, table):
    return _embed(x, table)


# speedup vs baseline: 1.0009x; 1.0009x over previous
import functools
import math

import jax
import jax.numpy as jnp
from jax import lax
from jax.experimental import pallas as pl
from jax.experimental.pallas import tpu as pltpu
from jax.experimental.pallas import tpu_sc as plsc

D = 128
L = 16
NC = 2
NS = 16
NW = NC * NS
SUB = 128
NBUF = 4
SCALE = float(math.sqrt(float(D)))


@jax.jit
def _embed(x, table):
    R, C = x.shape
    rows_per_w = R // NW
    n_chunks = rows_per_w
    b_per_w = rows_per_w * C

    mesh = plsc.VectorSubcoreMesh(core_axis_name="c", subcore_axis_name="s")

    scratch = [
        pltpu.VMEM((rows_per_w, C), jnp.int32),
    ]
    scratch += [pltpu.VMEM((C, D), jnp.float32) for _ in range(NBUF)]
    scratch += [pltpu.SemaphoreType.DMA for _ in range(3 * NBUF)]

    @functools.partial(
        pl.kernel,
        mesh=mesh,
        out_type=jax.ShapeDtypeStruct((R, C, D), jnp.float32),
        scratch_types=scratch,
    )
    def k(x_hbm, table_hbm, out_hbm, idx_v, *bufs_and_sems):
        rows = bufs_and_sems[:NBUF]
        gsem = bufs_and_sems[NBUF:2 * NBUF]
        g2sem = bufs_and_sems[2 * NBUF:3 * NBUF]
        ssem = bufs_and_sems[3 * NBUF:4 * NBUF]

        wid = lax.axis_index("s") * NC + lax.axis_index("c")
        xrow0 = wid * rows_per_w

        pltpu.sync_copy(x_hbm.at[pl.ds(xrow0, rows_per_w)], idx_v)

        def gather(g, b):
            return (
                pltpu.make_async_copy(
                    table_hbm.at[idx_v.at[g, pl.ds(0, SUB)]],
                    rows[b].at[pl.ds(0, SUB)],
                    gsem[b]),
                pltpu.make_async_copy(
                    table_hbm.at[idx_v.at[g, pl.ds(SUB, C - SUB)]],
                    rows[b].at[pl.ds(SUB, C - SUB)],
                    g2sem[b]),
            )

        def store_half(g, b, h):
            lo = 0 if h == 0 else SUB
            n = SUB if h == 0 else C - SUB
            return pltpu.make_async_copy(
                rows[b].at[pl.ds(lo, n)],
                out_hbm.at[xrow0 + g, pl.ds(lo, n)], ssem[b])

        def start_gather(g, b):
            d0, d1 = gather(g, b)
            d0.start()
            d1.start()

        def wait_gather(g, b):
            d0, d1 = gather(g, b)
            d0.wait()
            d1.wait()

        for b in range(NBUF - 2):
            start_gather(b, b)

        def round_body(go, carry):
            for b in range(NBUF):
                g = go * NBUF + b
                bp = (b - 2) % NBUF

                @pl.when(g + NBUF - 2 < n_chunks)
                def _():
                    @pl.when(g >= 2)
                    def _():
                        gm2 = lax.max(g - 2, 0)
                        store_half(gm2, bp, 0).wait()
                        store_half(gm2, bp, 1).wait()
                    start_gather(g + NBUF - 2, bp)

                d0, d1 = gather(g, b)

                def scale_quad(lo, hi):
                    def body(i, c2):
                        for r in range(4):
                            for v in range(D // L):
                                rows[b][i * 4 + r, pl.ds(v * L, L)] = (
                                    rows[b][i * 4 + r, pl.ds(v * L, L)]
                                    * SCALE
                                )
                        return c2
                    lax.fori_loop(lo // 4, hi // 4, body, 0)

                d0.wait()
                scale_quad(0, SUB)
                store_half(g, b, 0).start()
                d1.wait()
                scale_quad(SUB, C)
                store_half(g, b, 1).start()
            return carry

        lax.fori_loop(0, n_chunks // NBUF, round_body, 0)

        for b in range(NBUF):
            gl = n_chunks - NBUF + b
            store_half(gl, b, 0).wait()
            store_half(gl, b, 1).wait()

    return k(x, table)


def kernel(x, table):
    return _embed(x, table)


# speedup vs baseline: 1.0056x; 1.0047x over previous
import functools
import math

import jax
import jax.numpy as jnp
from jax import lax
from jax.experimental import pallas as pl
from jax.experimental.pallas import tpu as pltpu
from jax.experimental.pallas import tpu_sc as plsc

D = 128
L = 16
NC = 2
NS = 16
NW = NC * NS
H0 = 128
NBUF = 8
SCALE = float(math.sqrt(float(D)))


@jax.jit
def _embed(x, table):
    R, C = x.shape
    H1 = C - H0
    rows_per_w = R // NW
    n_chunks = 2 * rows_per_w
    assert n_chunks % NBUF == 0 and NBUF % 2 == 0

    mesh = plsc.VectorSubcoreMesh(core_axis_name="c", subcore_axis_name="s")

    scratch = [
        pltpu.VMEM((rows_per_w, C), jnp.int32),
    ]
    scratch += [
        pltpu.VMEM((H0 if b % 2 == 0 else H1, D), jnp.float32)
        for b in range(NBUF)
    ]
    scratch += [pltpu.SemaphoreType.DMA for _ in range(2 * NBUF)]

    @functools.partial(
        pl.kernel,
        mesh=mesh,
        out_type=jax.ShapeDtypeStruct((R, C, D), jnp.float32),
        scratch_types=scratch,
    )
    def k(x_hbm, table_hbm, out_hbm, idx_v, *bufs_and_sems):
        rows = bufs_and_sems[:NBUF]
        gsem = bufs_and_sems[NBUF:2 * NBUF]
        ssem = bufs_and_sems[2 * NBUF:3 * NBUF]

        wid = lax.axis_index("s") * NC + lax.axis_index("c")
        xrow0 = wid * rows_per_w

        pltpu.sync_copy(x_hbm.at[pl.ds(xrow0, rows_per_w)], idx_v)

        def halves(b):
            lo = 0 if b % 2 == 0 else H0
            n = H0 if b % 2 == 0 else H1
            return lo, n

        def gather(g, b):
            lo, n = halves(b)
            return pltpu.make_async_copy(
                table_hbm.at[idx_v.at[g // 2, pl.ds(lo, n)]],
                rows[b].at[pl.ds(0, n)],
                gsem[b])

        def store(g, b):
            lo, n = halves(b)
            return pltpu.make_async_copy(
                rows[b].at[pl.ds(0, n)],
                out_hbm.at[xrow0 + g // 2, pl.ds(lo, n)],
                ssem[b])

        for b in range(NBUF - 2):
            gather(b, b).start()

        def round_body(go, carry):
            for b in range(NBUF):
                g = go * NBUF + b
                bp = (b - 2) % NBUF

                @pl.when(g + NBUF - 2 < n_chunks)
                def _():
                    @pl.when(g >= 2)
                    def _():
                        store(lax.max(g - 2, 0), bp).wait()
                    gather(g + NBUF - 2, bp).start()

                gather(g, b).wait()

                _, n = halves(b)

                def scale_quad(i, c2):
                    for r in range(4):
                        for v in range(D // L):
                            rows[b][i * 4 + r, pl.ds(v * L, L)] = (
                                rows[b][i * 4 + r, pl.ds(v * L, L)] * SCALE
                            )
                    return c2

                lax.fori_loop(0, n // 4, scale_quad, 0)
                store(g, b).start()
            return carry

        lax.fori_loop(0, n_chunks // NBUF, round_body, 0)

        for b in range(NBUF):
            store(n_chunks - NBUF + b, b).wait()

    return k(x, table)


def kernel(x, table):
    return _embed(x, table)
